# Initial kernel scaffold; baseline (speedup 1.0000x reference)
import jax
import jax.numpy as jnp
from jax.experimental import pallas as pl


def kernel(feat0, feat1, feat2, feat3, feat4, rpn_conv_w, rpn_conv_b, rpn_cls_w, rpn_cls_b, rpn_box_w, rpn_box_b, fc1_w, fc1_b, fc2_w, fc2_b, cls_w, cls_b, reg_w, reg_b):
    raise NotImplementedError("write your pallas kernel here")


# speedup vs baseline: 1.8544x; 1.8544x over previous
import functools

import jax
import jax.numpy as jnp
import numpy as np
from jax.experimental import pallas as pl
from jax.experimental.pallas import tpu as pltpu

_B = 2
_C = 256
_NUM_CLASSES = 80
_IMG_H, _IMG_W = 512, 512
_STRIDES = [4, 8, 16, 32, 64]
_FEAT_HW = [(128, 128), (64, 64), (32, 32), (16, 16), (8, 8)]
_PRE_NMS_TOPK = 1000
_POST_NMS = 512
_IOU_THR = 0.7
_POOL = 7
_FC_DIM = 1024

_NMS_PAD = 4096
_NMS_SHAPE = (8, 512)
_OUT_SHAPE = (8, 64)


def _conv2d(x, w, b):
    out = jax.lax.conv_general_dilated(
        x, w, (1, 1), 'SAME', dimension_numbers=('NCHW', 'OIHW', 'NCHW'))
    return out + b[None, :, None, None]


def _make_anchors(hf, wf, stride):
    size = 8.0 * stride
    ratios = jnp.array([0.5, 1.0, 2.0], jnp.float32)
    hs = size * jnp.sqrt(ratios)
    ws = size / jnp.sqrt(ratios)
    cx = (jnp.arange(wf, dtype=jnp.float32) + 0.5) * stride
    cy = (jnp.arange(hf, dtype=jnp.float32) + 0.5) * stride
    cxg, cyg = jnp.meshgrid(cx, cy)
    x1 = cxg[..., None] - ws / 2.0
    y1 = cyg[..., None] - hs / 2.0
    x2 = cxg[..., None] + ws / 2.0
    y2 = cyg[..., None] + hs / 2.0
    return jnp.stack([x1, y1, x2, y2], -1).reshape(-1, 4)


def _decode(anchors, deltas):
    wa = anchors[:, 2] - anchors[:, 0]
    ha = anchors[:, 3] - anchors[:, 1]
    cxa = anchors[:, 0] + 0.5 * wa
    cya = anchors[:, 1] + 0.5 * ha
    dw = jnp.clip(deltas[:, 2], -4.135, 4.135)
    dh = jnp.clip(deltas[:, 3], -4.135, 4.135)
    cx = deltas[:, 0] * wa + cxa
    cy = deltas[:, 1] * ha + cya
    w = wa * jnp.exp(dw)
    h = ha * jnp.exp(dh)
    x1 = jnp.clip(cx - 0.5 * w, 0.0, _IMG_W)
    y1 = jnp.clip(cy - 0.5 * h, 0.0, _IMG_H)
    x2 = jnp.clip(cx + 0.5 * w, 0.0, _IMG_W)
    y2 = jnp.clip(cy + 0.5 * h, 0.0, _IMG_H)
    return jnp.stack([x1, y1, x2, y2], -1)


def _bilinear_sample(fmap, xs, ys):
    hf, wf = fmap.shape[1], fmap.shape[2]
    x0 = jnp.floor(xs)
    y0 = jnp.floor(ys)
    lx = xs - x0
    ly = ys - y0
    x0i = jnp.clip(x0.astype(jnp.int32), 0, wf - 1)
    x1i = jnp.clip(x0i + 1, 0, wf - 1)
    y0i = jnp.clip(y0.astype(jnp.int32), 0, hf - 1)
    y1i = jnp.clip(y0i + 1, 0, hf - 1)
    v00 = fmap[:, y0i, x0i]
    v01 = fmap[:, y0i, x1i]
    v10 = fmap[:, y1i, x0i]
    v11 = fmap[:, y1i, x1i]
    return (v00 * (1 - ly) * (1 - lx) + v01 * (1 - ly) * lx
            + v10 * ly * (1 - lx) + v11 * ly * lx)


def _roi_align_level(fmap, boxes, stride):
    def one(box):
        bx = box / stride
        bw = jnp.maximum(bx[2] - bx[0], 1e-2) / _POOL
        bh = jnp.maximum(bx[3] - bx[1], 1e-2) / _POOL
        xs = bx[0] + (jnp.arange(_POOL, dtype=jnp.float32) + 0.5) * bw
        ys = bx[1] + (jnp.arange(_POOL, dtype=jnp.float32) + 0.5) * bh
        xg, yg = jnp.meshgrid(xs, ys)
        vals = _bilinear_sample(fmap, xg.ravel() - 0.5, yg.ravel() - 0.5)
        return vals.reshape(fmap.shape[0], _POOL, _POOL)
    return jax.vmap(one)(boxes)


def _nms_kernel_body(sc_ref, x1_ref, y1_ref, x2_ref, y2_ref,
                     keep_ref, valid_ref):
    iota = (jax.lax.broadcasted_iota(jnp.int32, _NMS_SHAPE, 0) * 512
            + jax.lax.broadcasted_iota(jnp.int32, _NMS_SHAPE, 1))
    slot = (jax.lax.broadcasted_iota(jnp.int32, _OUT_SHAPE, 0) * 64
            + jax.lax.broadcasted_iota(jnp.int32, _OUT_SHAPE, 1))
    x1 = x1_ref[0]
    y1 = y1_ref[0]
    x2 = x2_ref[0]
    y2 = y2_ref[0]
    area = (x2 - x1) * (y2 - y1)

    def step(t, carry):
        sc, keep, valid = carry
        m = jnp.max(sc, keepdims=True)
        mask = sc == m
        idx = jnp.min(jnp.where(mask, iota, _NMS_PAD), keepdims=True)
        sel = iota == idx
        zero = jnp.zeros((), jnp.float32)
        bx1 = jnp.sum(jnp.where(sel, x1, zero), keepdims=True)
        by1 = jnp.sum(jnp.where(sel, y1, zero), keepdims=True)
        bx2 = jnp.sum(jnp.where(sel, x2, zero), keepdims=True)
        by2 = jnp.sum(jnp.where(sel, y2, zero), keepdims=True)
        ok = jnp.where(m > -1e30, 1.0, 0.0).astype(jnp.float32)
        ix1 = jnp.maximum(bx1, x1)
        iy1 = jnp.maximum(by1, y1)
        ix2 = jnp.minimum(bx2, x2)
        iy2 = jnp.minimum(by2, y2)
        inter = jnp.maximum(ix2 - ix1, 0.0) * jnp.maximum(iy2 - iy1, 0.0)
        a1 = (bx2 - bx1) * (by2 - by1)
        iou = inter / jnp.maximum(a1 + area - inter, 1e-6)
        sc = jnp.where(iou > _IOU_THR, jnp.float32(-1e31), sc)
        hot = slot == t
        keep = jnp.where(hot, idx, keep)
        valid = jnp.where(hot, ok, valid)
        return sc, keep, valid

    sc0 = sc_ref[0]
    keep0 = jnp.zeros(_OUT_SHAPE, jnp.int32)
    valid0 = jnp.zeros(_OUT_SHAPE, jnp.float32)
    _, keep, valid = jax.lax.fori_loop(0, _POST_NMS, step,
                                       (sc0, keep0, valid0))
    keep_ref[0] = keep
    valid_ref[0] = valid


def _nms_pallas(boxes, scores):
    n = scores.shape[1]
    pad = _NMS_PAD - n
    sc = jnp.concatenate(
        [scores, jnp.full((_B, pad), -3e31, jnp.float32)], axis=1)
    bx = jnp.concatenate(
        [boxes, jnp.zeros((_B, pad, 4), jnp.float32)], axis=1)
    sc = sc.reshape(_B, *_NMS_SHAPE)
    planes = [bx[:, :, i].reshape(_B, *_NMS_SHAPE) for i in range(4)]
    in_spec = pl.BlockSpec((1, *_NMS_SHAPE), lambda b: (b, 0, 0))
    out_spec = pl.BlockSpec((1, *_OUT_SHAPE), lambda b: (b, 0, 0))
    keep, valid = pl.pallas_call(
        _nms_kernel_body,
        grid=(_B,),
        in_specs=[in_spec] * 5,
        out_specs=[out_spec, out_spec],
        out_shape=[
            jax.ShapeDtypeStruct((_B, *_OUT_SHAPE), jnp.int32),
            jax.ShapeDtypeStruct((_B, *_OUT_SHAPE), jnp.float32),
        ],
    )(sc, *planes)
    return keep.reshape(_B, _POST_NMS), valid.reshape(_B, _POST_NMS)


def kernel(feat0, feat1, feat2, feat3, feat4, rpn_conv_w, rpn_conv_b,
           rpn_cls_w, rpn_cls_b, rpn_box_w, rpn_box_b, fc1_w, fc1_b,
           fc2_w, fc2_b, cls_w, cls_b, reg_w, reg_b):
    feats = [feat0, feat1, feat2, feat3, feat4]
    rpn_cls, rpn_box = [], []
    for f in feats:
        t = jax.nn.relu(_conv2d(f, rpn_conv_w, rpn_conv_b))
        rpn_cls.append(_conv2d(t, rpn_cls_w, rpn_cls_b))
        rpn_box.append(_conv2d(t, rpn_box_w, rpn_box_b))
    anchors = [_make_anchors(h, w, s)
               for (h, w), s in zip(_FEAT_HW, _STRIDES)]

    per_img_boxes, per_img_scores = [], []
    for b in range(_B):
        lvl_boxes, lvl_scores = [], []
        for l in range(5):
            cls = rpn_cls[l][b].transpose(1, 2, 0).reshape(-1)
            box = rpn_box[l][b].transpose(1, 2, 0).reshape(-1, 4)
            scores = jax.nn.sigmoid(cls)
            k = min(_PRE_NMS_TOPK, scores.shape[0])
            topv, topi = jax.lax.top_k(scores, k)
            lvl_boxes.append(_decode(anchors[l][topi], box[topi]))
            lvl_scores.append(topv)
        per_img_boxes.append(jnp.concatenate(lvl_boxes, 0))
        per_img_scores.append(jnp.concatenate(lvl_scores, 0))

    boxes_cat = jnp.stack(per_img_boxes, 0)
    scores_cat = jnp.stack(per_img_scores, 0)
    keep, valid = _nms_pallas(boxes_cat, scores_cat)

    all_boxes, all_scores = [], []
    for b in range(_B):
        vm = valid[b]
        all_boxes.append(boxes_cat[b][keep[b]] * vm[:, None])
        all_scores.append(scores_cat[b][keep[b]] * vm)

    roi_feats = []
    for b in range(_B):
        boxes_b = all_boxes[b]
        area = (boxes_b[:, 2] - boxes_b[:, 0]) * (boxes_b[:, 3] - boxes_b[:, 1])
        lvl = jnp.clip(jnp.floor(4.0 + jnp.log2(jnp.sqrt(area) / 224.0 + 1e-8)),
                       2, 5).astype(jnp.int32) - 2
        per_lvl = jnp.stack([_roi_align_level(feats[l][b], boxes_b, _STRIDES[l])
                             for l in range(4)], 0)
        roi_feats.append(per_lvl[lvl, jnp.arange(_POST_NMS)])
    roi = jnp.concatenate(roi_feats, 0).reshape(_B * _POST_NMS, -1)
    h1 = jax.nn.relu(roi @ fc1_w.T + fc1_b)
    h2 = jax.nn.relu(h1 @ fc2_w.T + fc2_b)
    cls_score = h2 @ cls_w.T + cls_b
    bbox_pred = h2 @ reg_w.T + reg_b
    return cls_score, bbox_pred, jnp.stack(all_boxes, 0), jnp.stack(all_scores, 0)


# speedup vs baseline: 12.3909x; 6.6818x over previous
import functools

import jax
import jax.numpy as jnp
import numpy as np
from jax.experimental import pallas as pl
from jax.experimental.pallas import tpu as pltpu
from jax.experimental.pallas import tpu_sc as plsc

_B = 2
_C = 256
_NUM_CLASSES = 80
_IMG_H, _IMG_W = 512, 512
_STRIDES = [4, 8, 16, 32, 64]
_FEAT_HW = [(128, 128), (64, 64), (32, 32), (16, 16), (8, 8)]
_PRE_NMS_TOPK = 1000
_POST_NMS = 512
_IOU_THR = 0.7
_POOL = 7
_FC_DIM = 1024

_NMS_PAD = 4096
_NMS_SHAPE = (8, 512)
_OUT_SHAPE = (8, 64)


_CONV_CHUNK = 128


def _level_plan(H, W):
    Wp = W + 2
    M = H * Wp
    n = max(1, -(-M // 4224))
    S = -(-M // (_CONV_CHUNK * n)) * _CONV_CHUNK
    need = 2 * Wp + 2
    halo = None
    for d in range(-(-need // 8) * 8, S + 8, 8):
        if S % d == 0:
            halo = d
            break
    nc = S // _CONV_CHUNK
    tailc = 1
    for c in range(nc):
        if c * _CONV_CHUNK + need + _CONV_CHUNK - 1 >= S:
            tailc = nc - c
            break
    return Wp, M, n, S, halo, nc, tailc


def _make_conv_call(H, W):
    Wp, M, n, S, HALO, NC, TAILC = _level_plan(H, W)
    TAIL = TAILC * _CONV_CHUNK
    offs = [ky * Wp + kx for ky in range(3) for kx in range(3)]

    def body(xc, xn, w9r, whr, cbr, hbr, outr, xs):
        xs[0:TAIL, :] = xc[0, S - TAIL:S, :]
        xs[TAIL:TAIL + HALO, :] = xn[0]
        cb = cbr[...]
        hb = hbr[...]

        def chunk(read_fn):
            acc = None
            for k in range(9):
                xk = read_fn(offs[k]).astype(jnp.bfloat16)
                p = jax.lax.dot_general(
                    xk, w9r[k], (((1,), (0,)), ((), ())),
                    preferred_element_type=jnp.float32)
                acc = p if acc is None else acc + p
            t = jnp.maximum(acc + cb, 0.0).astype(jnp.bfloat16)
            return jax.lax.dot_general(
                t, whr[...], (((1,), (0,)), ((), ())),
                preferred_element_type=jnp.float32) + hb

        for c in range(NC - TAILC):
            base = c * _CONV_CHUNK
            outr[0, pl.ds(base, _CONV_CHUNK), :] = chunk(
                lambda off, _b=base: xc[0, pl.ds(_b + off, _CONV_CHUNK), :])
        for ci in range(TAILC):
            c = NC - TAILC + ci
            res = chunk(
                lambda off, _ci=ci: xs[pl.ds(_ci * _CONV_CHUNK + off,
                                             _CONV_CHUNK), :])
            outr[0, pl.ds(c * _CONV_CHUNK, _CONV_CHUNK), :] = res

    rows = n * S + HALO
    call = pl.pallas_call(
        body,
        grid=(_B, n),
        in_specs=[
            pl.BlockSpec((1, S, _C), lambda b, i: (b, i, 0)),
            pl.BlockSpec((1, HALO, _C),
                         lambda b, i: (b, (i + 1) * (S // HALO), 0)),
            pl.BlockSpec((9, _C, _C), lambda b, i: (0, 0, 0)),
            pl.BlockSpec((_C, 128), lambda b, i: (0, 0)),
            pl.BlockSpec((1, _C), lambda b, i: (0, 0)),
            pl.BlockSpec((1, 128), lambda b, i: (0, 0)),
        ],
        out_specs=pl.BlockSpec((1, S, 128), lambda b, i: (b, i, 0)),
        out_shape=jax.ShapeDtypeStruct((_B, n * S, 128), jnp.float32),
        scratch_shapes=[pltpu.VMEM((TAIL + HALO, _C), jnp.float32)],
    )

    def run(f, w9, wh, cb, hb):
        x = jnp.pad(f.transpose(0, 2, 3, 1), ((0, 0), (1, 1), (1, 1), (0, 0)))
        x = x.reshape(_B, (H + 2) * Wp, _C)
        x = jnp.pad(x, ((0, 0), (0, rows - x.shape[1]), (0, 0)))
        out = call(x, x, w9, wh, cb, hb)
        out = out[:, :M, :].reshape(_B, H, Wp, 128)
        return out[:, :, :W, 0:3], out[:, :, :W, 3:15], x

    return run


_CONV_RUNS = [_make_conv_call(h, w) for h, w in _FEAT_HW]

_ROWS03, _CUM03 = [], []
_off = 0
for _hw in _FEAT_HW[:4]:
    _Wp, _M, _n, _S, _halo, _nc, _tailc = _level_plan(*_hw)
    _CUM03.append(_off)
    _ROWS03.append(_n * _S + _halo)
    _off += _B * _ROWS03[-1]
_ARENA_ROWS = _off

_NPTS = _B * _POST_NMS * _POOL * _POOL
_NIDX = 4 * _NPTS
_GW = 128


def _sc_gather(arena, idx):
    mesh = plsc.VectorSubcoreMesh(core_axis_name="c", subcore_axis_name="s")

    @pl.kernel(out_type=jax.ShapeDtypeStruct((_NIDX, _C), jnp.float32),
               mesh=mesh)
    def gk(x_hbm, i_hbm, o_hbm):
        def body(i_vmem, o_vmem):
            pltpu.sync_copy(x_hbm.at[i_vmem.at[0]], o_vmem)

        pltpu.emit_pipeline(
            body,
            grid=(_NIDX // _GW,),
            in_specs=[pl.BlockSpec((1, _GW), lambda i: (0, i))],
            out_specs=[pl.BlockSpec((_GW, _C), lambda i: (i, 0))],
            core_axis_name=("c", "s"),
            dimension_semantics=(pltpu.PARALLEL,),
        )(i_hbm, o_hbm)

    return gk(arena, idx.reshape(1, _NIDX))


_CMB_BLK = 1024


def _combine_body(g_ref, w_ref, o_ref):
    g = g_ref[...]
    w = w_ref[...]
    r = (g[0] * w[:, 0:1] + g[1] * w[:, 1:2]
         + g[2] * w[:, 2:3] + g[3] * w[:, 3:4])
    o_ref[...] = r.astype(jnp.bfloat16)


_combine_call = pl.pallas_call(
    _combine_body,
    grid=(_NPTS // _CMB_BLK,),
    in_specs=[
        pl.BlockSpec((4, _CMB_BLK, _C), lambda i: (0, i, 0)),
        pl.BlockSpec((_CMB_BLK, 4), lambda i: (i, 0)),
    ],
    out_specs=pl.BlockSpec((_CMB_BLK, _C), lambda i: (i, 0)),
    out_shape=jax.ShapeDtypeStruct((_NPTS, _C), jnp.bfloat16),
)


def _roi_indices(boxes):
    area = (boxes[..., 2] - boxes[..., 0]) * (boxes[..., 3] - boxes[..., 1])
    lvl = (jnp.clip(jnp.floor(4.0 + jnp.log2(jnp.sqrt(area) / 224.0 + 1e-8)),
                    2, 5).astype(jnp.int32) - 2)
    stride = jnp.array([4., 8., 16., 32.], jnp.float32)[lvl]
    wf = jnp.array([128, 64, 32, 16], jnp.int32)[lvl]
    wp = jnp.array([130, 66, 34, 18], jnp.int32)[lvl]
    bimg = jnp.arange(_B, dtype=jnp.int32)[:, None]
    base = (jnp.array(_CUM03, jnp.int32)[lvl]
            + bimg * jnp.array(_ROWS03, jnp.int32)[lvl])
    bx = boxes / stride[..., None]
    bw = jnp.maximum(bx[..., 2] - bx[..., 0], 1e-2) / _POOL
    bh = jnp.maximum(bx[..., 3] - bx[..., 1], 1e-2) / _POOL
    rng = jnp.arange(_POOL, dtype=jnp.float32) + 0.5
    xs = bx[..., 0:1] + rng * bw[..., None] - 0.5
    ys = bx[..., 1:2] + rng * bh[..., None] - 0.5
    x0 = jnp.floor(xs)
    lx = xs - x0
    y0 = jnp.floor(ys)
    ly = ys - y0
    wfm = (wf - 1)[..., None]
    x0i = jnp.clip(x0.astype(jnp.int32), 0, wfm)
    x1i = jnp.clip(x0i + 1, 0, wfm)
    y0i = jnp.clip(y0.astype(jnp.int32), 0, wfm)
    y1i = jnp.clip(y0i + 1, 0, wfm)
    fy0 = (y0i + 1) * wp[..., None]
    fy1 = (y1i + 1) * wp[..., None]
    fx0 = x0i + 1
    fx1 = x1i + 1

    def mk(fy, fx):
        return (base[..., None, None] + fy[..., :, None]
                + fx[..., None, :]).reshape(-1)

    idx = jnp.concatenate([mk(fy0, fx0), mk(fy0, fx1),
                           mk(fy1, fx0), mk(fy1, fx1)], 0)

    def wmk(a, c):
        return (a[..., :, None] * c[..., None, :]).reshape(-1)

    w4 = jnp.stack([wmk(1.0 - ly, 1.0 - lx), wmk(1.0 - ly, lx),
                    wmk(ly, 1.0 - lx), wmk(ly, lx)], 1)
    return idx, w4


_FC_MBLK = 128
_FC_KC = 1568


def _fc_body(roi_ref, w1_ref, b1_ref, w2_ref, b2_ref, wc_ref, bc_ref,
             wr_ref, br_ref, oc_ref, orr_ref):
    acc = None
    for k in range(_C * _POOL * _POOL // _FC_KC):
        a = roi_ref[:, pl.ds(k * _FC_KC, _FC_KC)]
        wchunk = w1_ref[pl.ds(k * _FC_KC, _FC_KC), :]
        p = jax.lax.dot_general(a, wchunk, (((1,), (0,)), ((), ())),
                                preferred_element_type=jnp.float32)
        acc = p if acc is None else acc + p
    h1 = jnp.maximum(acc + b1_ref[...], 0.0).astype(jnp.bfloat16)
    h2f = jax.lax.dot_general(h1, w2_ref[...], (((1,), (0,)), ((), ())),
                              preferred_element_type=jnp.float32) + b2_ref[...]
    h2 = jnp.maximum(h2f, 0.0).astype(jnp.bfloat16)
    oc_ref[...] = jax.lax.dot_general(
        h2, wc_ref[...], (((1,), (0,)), ((), ())),
        preferred_element_type=jnp.float32) + bc_ref[...]
    orr_ref[...] = jax.lax.dot_general(
        h2, wr_ref[...], (((1,), (0,)), ((), ())),
        preferred_element_type=jnp.float32) + br_ref[...]


_fc_call = pl.pallas_call(
    _fc_body,
    grid=(_B * _POST_NMS // _FC_MBLK,),
    in_specs=[
        pl.BlockSpec((_FC_MBLK, _C * _POOL * _POOL), lambda i: (i, 0)),
        pl.BlockSpec((_C * _POOL * _POOL, _FC_DIM), lambda i: (0, 0)),
        pl.BlockSpec((1, _FC_DIM), lambda i: (0, 0)),
        pl.BlockSpec((_FC_DIM, _FC_DIM), lambda i: (0, 0)),
        pl.BlockSpec((1, _FC_DIM), lambda i: (0, 0)),
        pl.BlockSpec((_FC_DIM, 128), lambda i: (0, 0)),
        pl.BlockSpec((1, 128), lambda i: (0, 0)),
        pl.BlockSpec((_FC_DIM, 384), lambda i: (0, 0)),
        pl.BlockSpec((1, 384), lambda i: (0, 0)),
    ],
    out_specs=[
        pl.BlockSpec((_FC_MBLK, 128), lambda i: (i, 0)),
        pl.BlockSpec((_FC_MBLK, 384), lambda i: (i, 0)),
    ],
    out_shape=[
        jax.ShapeDtypeStruct((_B * _POST_NMS, 128), jnp.float32),
        jax.ShapeDtypeStruct((_B * _POST_NMS, 384), jnp.float32),
    ],
)


def _make_anchors(hf, wf, stride):
    size = 8.0 * stride
    ratios = jnp.array([0.5, 1.0, 2.0], jnp.float32)
    hs = size * jnp.sqrt(ratios)
    ws = size / jnp.sqrt(ratios)
    cx = (jnp.arange(wf, dtype=jnp.float32) + 0.5) * stride
    cy = (jnp.arange(hf, dtype=jnp.float32) + 0.5) * stride
    cxg, cyg = jnp.meshgrid(cx, cy)
    x1 = cxg[..., None] - ws / 2.0
    y1 = cyg[..., None] - hs / 2.0
    x2 = cxg[..., None] + ws / 2.0
    y2 = cyg[..., None] + hs / 2.0
    return jnp.stack([x1, y1, x2, y2], -1).reshape(-1, 4)


def _decode(anchors, deltas):
    wa = anchors[:, 2] - anchors[:, 0]
    ha = anchors[:, 3] - anchors[:, 1]
    cxa = anchors[:, 0] + 0.5 * wa
    cya = anchors[:, 1] + 0.5 * ha
    dw = jnp.clip(deltas[:, 2], -4.135, 4.135)
    dh = jnp.clip(deltas[:, 3], -4.135, 4.135)
    cx = deltas[:, 0] * wa + cxa
    cy = deltas[:, 1] * ha + cya
    w = wa * jnp.exp(dw)
    h = ha * jnp.exp(dh)
    x1 = jnp.clip(cx - 0.5 * w, 0.0, _IMG_W)
    y1 = jnp.clip(cy - 0.5 * h, 0.0, _IMG_H)
    x2 = jnp.clip(cx + 0.5 * w, 0.0, _IMG_W)
    y2 = jnp.clip(cy + 0.5 * h, 0.0, _IMG_H)
    return jnp.stack([x1, y1, x2, y2], -1)


def _bilinear_sample(fmap, xs, ys):
    hf, wf = fmap.shape[1], fmap.shape[2]
    x0 = jnp.floor(xs)
    y0 = jnp.floor(ys)
    lx = xs - x0
    ly = ys - y0
    x0i = jnp.clip(x0.astype(jnp.int32), 0, wf - 1)
    x1i = jnp.clip(x0i + 1, 0, wf - 1)
    y0i = jnp.clip(y0.astype(jnp.int32), 0, hf - 1)
    y1i = jnp.clip(y0i + 1, 0, hf - 1)
    v00 = fmap[:, y0i, x0i]
    v01 = fmap[:, y0i, x1i]
    v10 = fmap[:, y1i, x0i]
    v11 = fmap[:, y1i, x1i]
    return (v00 * (1 - ly) * (1 - lx) + v01 * (1 - ly) * lx
            + v10 * ly * (1 - lx) + v11 * ly * lx)


def _roi_align_level(fmap, boxes, stride):
    def one(box):
        bx = box / stride
        bw = jnp.maximum(bx[2] - bx[0], 1e-2) / _POOL
        bh = jnp.maximum(bx[3] - bx[1], 1e-2) / _POOL
        xs = bx[0] + (jnp.arange(_POOL, dtype=jnp.float32) + 0.5) * bw
        ys = bx[1] + (jnp.arange(_POOL, dtype=jnp.float32) + 0.5) * bh
        xg, yg = jnp.meshgrid(xs, ys)
        vals = _bilinear_sample(fmap, xg.ravel() - 0.5, yg.ravel() - 0.5)
        return vals.reshape(fmap.shape[0], _POOL, _POOL)
    return jax.vmap(one)(boxes)


def _nms_kernel_body(sc_ref, x1_ref, y1_ref, x2_ref, y2_ref,
                     keep_ref, valid_ref):
    iota = (jax.lax.broadcasted_iota(jnp.int32, _NMS_SHAPE, 0) * 512
            + jax.lax.broadcasted_iota(jnp.int32, _NMS_SHAPE, 1))
    slot = (jax.lax.broadcasted_iota(jnp.int32, _OUT_SHAPE, 0) * 64
            + jax.lax.broadcasted_iota(jnp.int32, _OUT_SHAPE, 1))
    x1 = x1_ref[0]
    y1 = y1_ref[0]
    x2 = x2_ref[0]
    y2 = y2_ref[0]
    area = (x2 - x1) * (y2 - y1)

    def step(t, carry):
        sc, keep, valid = carry
        m = jnp.max(sc, keepdims=True)
        mask = sc == m
        idx = jnp.min(jnp.where(mask, iota, _NMS_PAD), keepdims=True)
        sel = iota == idx
        zero = jnp.zeros((), jnp.float32)
        bx1 = jnp.sum(jnp.where(sel, x1, zero), keepdims=True)
        by1 = jnp.sum(jnp.where(sel, y1, zero), keepdims=True)
        bx2 = jnp.sum(jnp.where(sel, x2, zero), keepdims=True)
        by2 = jnp.sum(jnp.where(sel, y2, zero), keepdims=True)
        ok = jnp.where(m > -1e30, 1.0, 0.0).astype(jnp.float32)
        ix1 = jnp.maximum(bx1, x1)
        iy1 = jnp.maximum(by1, y1)
        ix2 = jnp.minimum(bx2, x2)
        iy2 = jnp.minimum(by2, y2)
        inter = jnp.maximum(ix2 - ix1, 0.0) * jnp.maximum(iy2 - iy1, 0.0)
        a1 = (bx2 - bx1) * (by2 - by1)
        iou = inter / jnp.maximum(a1 + area - inter, 1e-6)
        sc = jnp.where(iou > _IOU_THR, jnp.float32(-1e31), sc)
        hot = slot == t
        keep = jnp.where(hot, idx, keep)
        valid = jnp.where(hot, ok, valid)
        return sc, keep, valid

    sc0 = sc_ref[0]
    keep0 = jnp.zeros(_OUT_SHAPE, jnp.int32)
    valid0 = jnp.zeros(_OUT_SHAPE, jnp.float32)
    _, keep, valid = jax.lax.fori_loop(0, _POST_NMS, step,
                                       (sc0, keep0, valid0))
    keep_ref[0] = keep
    valid_ref[0] = valid


def _nms_pallas(boxes, scores):
    n = scores.shape[1]
    pad = _NMS_PAD - n
    sc = jnp.concatenate(
        [scores, jnp.full((_B, pad), -3e31, jnp.float32)], axis=1)
    bx = jnp.concatenate(
        [boxes, jnp.zeros((_B, pad, 4), jnp.float32)], axis=1)
    sc = sc.reshape(_B, *_NMS_SHAPE)
    planes = [bx[:, :, i].reshape(_B, *_NMS_SHAPE) for i in range(4)]
    in_spec = pl.BlockSpec((1, *_NMS_SHAPE), lambda b: (b, 0, 0))
    out_spec = pl.BlockSpec((1, *_OUT_SHAPE), lambda b: (b, 0, 0))
    keep, valid = pl.pallas_call(
        _nms_kernel_body,
        grid=(_B,),
        in_specs=[in_spec] * 5,
        out_specs=[out_spec, out_spec],
        out_shape=[
            jax.ShapeDtypeStruct((_B, *_OUT_SHAPE), jnp.int32),
            jax.ShapeDtypeStruct((_B, *_OUT_SHAPE), jnp.float32),
        ],
    )(sc, *planes)
    return keep.reshape(_B, _POST_NMS), valid.reshape(_B, _POST_NMS)


def kernel(feat0, feat1, feat2, feat3, feat4, rpn_conv_w, rpn_conv_b,
           rpn_cls_w, rpn_cls_b, rpn_box_w, rpn_box_b, fc1_w, fc1_b,
           fc2_w, fc2_b, cls_w, cls_b, reg_w, reg_b):
    feats = [feat0, feat1, feat2, feat3, feat4]
    w9 = rpn_conv_w.transpose(2, 3, 1, 0).reshape(9, _C, _C).astype(jnp.bfloat16)
    wh = jnp.concatenate([rpn_cls_w[:, :, 0, 0], rpn_box_w[:, :, 0, 0]], 0).T
    wh = jnp.pad(wh, ((0, 0), (0, 128 - 15))).astype(jnp.bfloat16)
    hb = jnp.pad(jnp.concatenate([rpn_cls_b, rpn_box_b]), (0, 128 - 15))[None]
    cb = rpn_conv_b[None]
    rpn_cls, rpn_box, flats = [], [], []
    for l, f in enumerate(feats):
        cls_l, box_l, xf_l = _CONV_RUNS[l](f, w9, wh, cb, hb)
        rpn_cls.append(cls_l)
        rpn_box.append(box_l)
        flats.append(xf_l)
    anchors = [_make_anchors(h, w, s)
               for (h, w), s in zip(_FEAT_HW, _STRIDES)]

    per_img_boxes, per_img_scores = [], []
    for b in range(_B):
        lvl_boxes, lvl_scores = [], []
        for l in range(5):
            cls = rpn_cls[l][b].reshape(-1)
            box = rpn_box[l][b].reshape(-1, 4)
            scores = jax.nn.sigmoid(cls)
            k = min(_PRE_NMS_TOPK, scores.shape[0])
            topv, topi = jax.lax.top_k(scores, k)
            lvl_boxes.append(_decode(anchors[l][topi], box[topi]))
            lvl_scores.append(topv)
        per_img_boxes.append(jnp.concatenate(lvl_boxes, 0))
        per_img_scores.append(jnp.concatenate(lvl_scores, 0))

    boxes_cat = jnp.stack(per_img_boxes, 0)
    scores_cat = jnp.stack(per_img_scores, 0)
    keep, valid = _nms_pallas(boxes_cat, scores_cat)

    all_boxes, all_scores = [], []
    for b in range(_B):
        vm = valid[b]
        all_boxes.append(boxes_cat[b][keep[b]] * vm[:, None])
        all_scores.append(scores_cat[b][keep[b]] * vm)

    boxes_all = jnp.stack(all_boxes, 0)
    idx, w4 = _roi_indices(boxes_all)
    arena = jnp.concatenate([f.reshape(-1, _C) for f in flats[:4]], 0)
    g = _sc_gather(arena, idx)
    roi_bf = _combine_call(g.reshape(4, _NPTS, _C), w4)
    roi2 = roi_bf.reshape(_B * _POST_NMS, _POOL * _POOL * _C)
    w1p = (fc1_w.reshape(_FC_DIM, _C, _POOL, _POOL).transpose(0, 2, 3, 1)
           .reshape(_FC_DIM, -1).T.astype(jnp.bfloat16))
    w2t = fc2_w.T.astype(jnp.bfloat16)
    wct = jnp.pad(cls_w, ((0, 128 - 81), (0, 0))).T.astype(jnp.bfloat16)
    wrt = jnp.pad(reg_w, ((0, 384 - 320), (0, 0))).T.astype(jnp.bfloat16)
    bc = jnp.pad(cls_b, (0, 128 - 81))[None]
    br = jnp.pad(reg_b, (0, 384 - 320))[None]
    oc, orr = _fc_call(roi2, w1p, fc1_b[None], w2t, fc2_b[None],
                       wct, bc, wrt, br)
    cls_score = oc[:, :_NUM_CLASSES + 1]
    bbox_pred = orr[:, :_NUM_CLASSES * 4]
    return cls_score, bbox_pred, boxes_all, jnp.stack(all_scores, 0)


# speedup vs baseline: 12.8199x; 1.0346x over previous
import functools

import jax
import jax.numpy as jnp
import numpy as np
from jax.experimental import pallas as pl
from jax.experimental.pallas import tpu as pltpu
from jax.experimental.pallas import tpu_sc as plsc

_B = 2
_C = 256
_NUM_CLASSES = 80
_IMG_H, _IMG_W = 512, 512
_STRIDES = [4, 8, 16, 32, 64]
_FEAT_HW = [(128, 128), (64, 64), (32, 32), (16, 16), (8, 8)]
_PRE_NMS_TOPK = 1000
_POST_NMS = 512
_IOU_THR = 0.7
_POOL = 7
_FC_DIM = 1024

_NMS_PAD = 4096
_NMS_SHAPE = (8, 512)
_OUT_SHAPE = (8, 64)


_CONV_CHUNK = 128


def _level_plan(H, W):
    Wp = W + 2
    M = H * Wp
    n = max(1, -(-M // 4224))
    S = -(-M // (_CONV_CHUNK * n)) * _CONV_CHUNK
    need = 2 * Wp + 2
    halo = None
    for d in range(-(-need // 8) * 8, S + 8, 8):
        if S % d == 0:
            halo = d
            break
    nc = S // _CONV_CHUNK
    tailc = 1
    for c in range(nc):
        if c * _CONV_CHUNK + need + _CONV_CHUNK - 1 >= S:
            tailc = nc - c
            break
    return Wp, M, n, S, halo, nc, tailc


def _make_conv_call(H, W):
    Wp, M, n, S, HALO, NC, TAILC = _level_plan(H, W)
    TAIL = TAILC * _CONV_CHUNK
    offs = [ky * Wp + kx for ky in range(3) for kx in range(3)]

    def body(xc, xn, w9r, whr, cbr, hbr, outr, xs):
        xs[0:S, :] = xc[0].astype(jnp.bfloat16)
        xs[S:S + HALO, :] = xn[0].astype(jnp.bfloat16)
        cb = cbr[...]
        hb = hbr[...]

        def chunk(base):
            acc = None
            for k in range(9):
                xk = xs[pl.ds(base + offs[k], _CONV_CHUNK), :]
                p = jax.lax.dot_general(
                    xk, w9r[k], (((1,), (0,)), ((), ())),
                    preferred_element_type=jnp.float32)
                acc = p if acc is None else acc + p
            t = jnp.maximum(acc + cb, 0.0).astype(jnp.bfloat16)
            return jax.lax.dot_general(
                t, whr[...], (((1,), (0,)), ((), ())),
                preferred_element_type=jnp.float32) + hb

        for c in range(NC):
            base = c * _CONV_CHUNK
            outr[0, pl.ds(base, _CONV_CHUNK), :] = chunk(base)

    rows = n * S + HALO
    call = pl.pallas_call(
        body,
        grid=(_B, n),
        in_specs=[
            pl.BlockSpec((1, S, _C), lambda b, i: (b, i, 0)),
            pl.BlockSpec((1, HALO, _C),
                         lambda b, i: (b, (i + 1) * (S // HALO), 0)),
            pl.BlockSpec((9, _C, _C), lambda b, i: (0, 0, 0)),
            pl.BlockSpec((_C, 128), lambda b, i: (0, 0)),
            pl.BlockSpec((1, _C), lambda b, i: (0, 0)),
            pl.BlockSpec((1, 128), lambda b, i: (0, 0)),
        ],
        out_specs=pl.BlockSpec((1, S, 128), lambda b, i: (b, i, 0)),
        out_shape=jax.ShapeDtypeStruct((_B, n * S, 128), jnp.float32),
        scratch_shapes=[pltpu.VMEM((S + HALO, _C), jnp.bfloat16)],
    )

    def run(f, w9, wh, cb, hb):
        x = jnp.pad(f.transpose(0, 2, 3, 1), ((0, 0), (1, 1), (1, 1), (0, 0)))
        x = x.reshape(_B, (H + 2) * Wp, _C)
        x = jnp.pad(x, ((0, 0), (0, rows - x.shape[1]), (0, 0)))
        out = call(x, x, w9, wh, cb, hb)
        out = out[:, :M, :].reshape(_B, H, Wp, 128)
        return out[:, :, :W, 0:3], out[:, :, :W, 3:15], x

    return run


_CONV_RUNS = [_make_conv_call(h, w) for h, w in _FEAT_HW]

_ROWS03, _CUM03 = [], []
_off = 0
for _hw in _FEAT_HW[:4]:
    _Wp, _M, _n, _S, _halo, _nc, _tailc = _level_plan(*_hw)
    _CUM03.append(_off)
    _ROWS03.append(_n * _S + _halo)
    _off += _B * _ROWS03[-1]
_ARENA_ROWS = _off

_NPTS = _B * _POST_NMS * _POOL * _POOL
_NIDX = 4 * _NPTS
_GW = 128


def _sc_gather(arena, idx):
    mesh = plsc.VectorSubcoreMesh(core_axis_name="c", subcore_axis_name="s")

    @pl.kernel(out_type=jax.ShapeDtypeStruct((_NIDX, _C), jnp.float32),
               mesh=mesh)
    def gk(x_hbm, i_hbm, o_hbm):
        def body(i_vmem, o_vmem):
            pltpu.sync_copy(x_hbm.at[i_vmem.at[0]], o_vmem)

        pltpu.emit_pipeline(
            body,
            grid=(_NIDX // _GW,),
            in_specs=[pl.BlockSpec((1, _GW), lambda i: (0, i))],
            out_specs=[pl.BlockSpec((_GW, _C), lambda i: (i, 0))],
            core_axis_name=("c", "s"),
            dimension_semantics=(pltpu.PARALLEL,),
        )(i_hbm, o_hbm)

    return gk(arena, idx.reshape(1, _NIDX))


_CMB_BLK = 1024


def _combine_body(g_ref, w_ref, o_ref):
    g = g_ref[...]
    w = w_ref[...]
    r = (g[0] * w[:, 0:1] + g[1] * w[:, 1:2]
         + g[2] * w[:, 2:3] + g[3] * w[:, 3:4])
    o_ref[...] = r.astype(jnp.bfloat16)


_combine_call = pl.pallas_call(
    _combine_body,
    grid=(_NPTS // _CMB_BLK,),
    in_specs=[
        pl.BlockSpec((4, _CMB_BLK, _C), lambda i: (0, i, 0)),
        pl.BlockSpec((_CMB_BLK, 4), lambda i: (i, 0)),
    ],
    out_specs=pl.BlockSpec((_CMB_BLK, _C), lambda i: (i, 0)),
    out_shape=jax.ShapeDtypeStruct((_NPTS, _C), jnp.bfloat16),
)


def _roi_indices(boxes):
    area = (boxes[..., 2] - boxes[..., 0]) * (boxes[..., 3] - boxes[..., 1])
    lvl = (jnp.clip(jnp.floor(4.0 + jnp.log2(jnp.sqrt(area) / 224.0 + 1e-8)),
                    2, 5).astype(jnp.int32) - 2)
    stride = jnp.array([4., 8., 16., 32.], jnp.float32)[lvl]
    wf = jnp.array([128, 64, 32, 16], jnp.int32)[lvl]
    wp = jnp.array([130, 66, 34, 18], jnp.int32)[lvl]
    bimg = jnp.arange(_B, dtype=jnp.int32)[:, None]
    base = (jnp.array(_CUM03, jnp.int32)[lvl]
            + bimg * jnp.array(_ROWS03, jnp.int32)[lvl])
    bx = boxes / stride[..., None]
    bw = jnp.maximum(bx[..., 2] - bx[..., 0], 1e-2) / _POOL
    bh = jnp.maximum(bx[..., 3] - bx[..., 1], 1e-2) / _POOL
    rng = jnp.arange(_POOL, dtype=jnp.float32) + 0.5
    xs = bx[..., 0:1] + rng * bw[..., None] - 0.5
    ys = bx[..., 1:2] + rng * bh[..., None] - 0.5
    x0 = jnp.floor(xs)
    lx = xs - x0
    y0 = jnp.floor(ys)
    ly = ys - y0
    wfm = (wf - 1)[..., None]
    x0i = jnp.clip(x0.astype(jnp.int32), 0, wfm)
    x1i = jnp.clip(x0i + 1, 0, wfm)
    y0i = jnp.clip(y0.astype(jnp.int32), 0, wfm)
    y1i = jnp.clip(y0i + 1, 0, wfm)
    fy0 = (y0i + 1) * wp[..., None]
    fy1 = (y1i + 1) * wp[..., None]
    fx0 = x0i + 1
    fx1 = x1i + 1

    def mk(fy, fx):
        return (base[..., None, None] + fy[..., :, None]
                + fx[..., None, :]).reshape(-1)

    idx = jnp.concatenate([mk(fy0, fx0), mk(fy0, fx1),
                           mk(fy1, fx0), mk(fy1, fx1)], 0)

    def wmk(a, c):
        return (a[..., :, None] * c[..., None, :]).reshape(-1)

    w4 = jnp.stack([wmk(1.0 - ly, 1.0 - lx), wmk(1.0 - ly, lx),
                    wmk(ly, 1.0 - lx), wmk(ly, lx)], 1)
    return idx, w4


_FC_MBLK = 128
_FC_KC = 1568


def _fc_body(roi_ref, w1_ref, b1_ref, w2_ref, b2_ref, wc_ref, bc_ref,
             wr_ref, br_ref, oc_ref, orr_ref):
    acc = None
    for k in range(_C * _POOL * _POOL // _FC_KC):
        a = roi_ref[:, pl.ds(k * _FC_KC, _FC_KC)]
        wchunk = w1_ref[pl.ds(k * _FC_KC, _FC_KC), :]
        p = jax.lax.dot_general(a, wchunk, (((1,), (0,)), ((), ())),
                                preferred_element_type=jnp.float32)
        acc = p if acc is None else acc + p
    h1 = jnp.maximum(acc + b1_ref[...], 0.0).astype(jnp.bfloat16)
    h2f = jax.lax.dot_general(h1, w2_ref[...], (((1,), (0,)), ((), ())),
                              preferred_element_type=jnp.float32) + b2_ref[...]
    h2 = jnp.maximum(h2f, 0.0).astype(jnp.bfloat16)
    oc_ref[...] = jax.lax.dot_general(
        h2, wc_ref[...], (((1,), (0,)), ((), ())),
        preferred_element_type=jnp.float32) + bc_ref[...]
    orr_ref[...] = jax.lax.dot_general(
        h2, wr_ref[...], (((1,), (0,)), ((), ())),
        preferred_element_type=jnp.float32) + br_ref[...]


_fc_call = pl.pallas_call(
    _fc_body,
    grid=(_B * _POST_NMS // _FC_MBLK,),
    in_specs=[
        pl.BlockSpec((_FC_MBLK, _C * _POOL * _POOL), lambda i: (i, 0)),
        pl.BlockSpec((_C * _POOL * _POOL, _FC_DIM), lambda i: (0, 0)),
        pl.BlockSpec((1, _FC_DIM), lambda i: (0, 0)),
        pl.BlockSpec((_FC_DIM, _FC_DIM), lambda i: (0, 0)),
        pl.BlockSpec((1, _FC_DIM), lambda i: (0, 0)),
        pl.BlockSpec((_FC_DIM, 128), lambda i: (0, 0)),
        pl.BlockSpec((1, 128), lambda i: (0, 0)),
        pl.BlockSpec((_FC_DIM, 384), lambda i: (0, 0)),
        pl.BlockSpec((1, 384), lambda i: (0, 0)),
    ],
    out_specs=[
        pl.BlockSpec((_FC_MBLK, 128), lambda i: (i, 0)),
        pl.BlockSpec((_FC_MBLK, 384), lambda i: (i, 0)),
    ],
    out_shape=[
        jax.ShapeDtypeStruct((_B * _POST_NMS, 128), jnp.float32),
        jax.ShapeDtypeStruct((_B * _POST_NMS, 384), jnp.float32),
    ],
)


def _make_anchors(hf, wf, stride):
    size = 8.0 * stride
    ratios = jnp.array([0.5, 1.0, 2.0], jnp.float32)
    hs = size * jnp.sqrt(ratios)
    ws = size / jnp.sqrt(ratios)
    cx = (jnp.arange(wf, dtype=jnp.float32) + 0.5) * stride
    cy = (jnp.arange(hf, dtype=jnp.float32) + 0.5) * stride
    cxg, cyg = jnp.meshgrid(cx, cy)
    x1 = cxg[..., None] - ws / 2.0
    y1 = cyg[..., None] - hs / 2.0
    x2 = cxg[..., None] + ws / 2.0
    y2 = cyg[..., None] + hs / 2.0
    return jnp.stack([x1, y1, x2, y2], -1).reshape(-1, 4)


def _decode(anchors, deltas):
    wa = anchors[:, 2] - anchors[:, 0]
    ha = anchors[:, 3] - anchors[:, 1]
    cxa = anchors[:, 0] + 0.5 * wa
    cya = anchors[:, 1] + 0.5 * ha
    dw = jnp.clip(deltas[:, 2], -4.135, 4.135)
    dh = jnp.clip(deltas[:, 3], -4.135, 4.135)
    cx = deltas[:, 0] * wa + cxa
    cy = deltas[:, 1] * ha + cya
    w = wa * jnp.exp(dw)
    h = ha * jnp.exp(dh)
    x1 = jnp.clip(cx - 0.5 * w, 0.0, _IMG_W)
    y1 = jnp.clip(cy - 0.5 * h, 0.0, _IMG_H)
    x2 = jnp.clip(cx + 0.5 * w, 0.0, _IMG_W)
    y2 = jnp.clip(cy + 0.5 * h, 0.0, _IMG_H)
    return jnp.stack([x1, y1, x2, y2], -1)


def _bilinear_sample(fmap, xs, ys):
    hf, wf = fmap.shape[1], fmap.shape[2]
    x0 = jnp.floor(xs)
    y0 = jnp.floor(ys)
    lx = xs - x0
    ly = ys - y0
    x0i = jnp.clip(x0.astype(jnp.int32), 0, wf - 1)
    x1i = jnp.clip(x0i + 1, 0, wf - 1)
    y0i = jnp.clip(y0.astype(jnp.int32), 0, hf - 1)
    y1i = jnp.clip(y0i + 1, 0, hf - 1)
    v00 = fmap[:, y0i, x0i]
    v01 = fmap[:, y0i, x1i]
    v10 = fmap[:, y1i, x0i]
    v11 = fmap[:, y1i, x1i]
    return (v00 * (1 - ly) * (1 - lx) + v01 * (1 - ly) * lx
            + v10 * ly * (1 - lx) + v11 * ly * lx)


def _roi_align_level(fmap, boxes, stride):
    def one(box):
        bx = box / stride
        bw = jnp.maximum(bx[2] - bx[0], 1e-2) / _POOL
        bh = jnp.maximum(bx[3] - bx[1], 1e-2) / _POOL
        xs = bx[0] + (jnp.arange(_POOL, dtype=jnp.float32) + 0.5) * bw
        ys = bx[1] + (jnp.arange(_POOL, dtype=jnp.float32) + 0.5) * bh
        xg, yg = jnp.meshgrid(xs, ys)
        vals = _bilinear_sample(fmap, xg.ravel() - 0.5, yg.ravel() - 0.5)
        return vals.reshape(fmap.shape[0], _POOL, _POOL)
    return jax.vmap(one)(boxes)


def _nms_kernel_body(sc_ref, x1_ref, y1_ref, x2_ref, y2_ref,
                     keep_ref, valid_ref):
    iota = (jax.lax.broadcasted_iota(jnp.int32, _NMS_SHAPE, 0) * 512
            + jax.lax.broadcasted_iota(jnp.int32, _NMS_SHAPE, 1))
    slot = (jax.lax.broadcasted_iota(jnp.int32, _OUT_SHAPE, 0) * 64
            + jax.lax.broadcasted_iota(jnp.int32, _OUT_SHAPE, 1))
    coords = [(x1_ref[b], y1_ref[b], x2_ref[b], y2_ref[b])
              for b in range(_B)]
    areas = [(x2 - x1) * (y2 - y1) for (x1, y1, x2, y2) in coords]

    def one_step(t, sc, keep, valid, b):
        x1, y1, x2, y2 = coords[b]
        area = areas[b]
        m = jnp.max(sc, keepdims=True)
        mask = sc == m
        idx = jnp.min(jnp.where(mask, iota, _NMS_PAD), keepdims=True)
        sel = iota == idx
        zero = jnp.zeros((), jnp.float32)
        bx1 = jnp.sum(jnp.where(sel, x1, zero), keepdims=True)
        by1 = jnp.sum(jnp.where(sel, y1, zero), keepdims=True)
        bx2 = jnp.sum(jnp.where(sel, x2, zero), keepdims=True)
        by2 = jnp.sum(jnp.where(sel, y2, zero), keepdims=True)
        ok = jnp.where(m > -1e30, 1.0, 0.0).astype(jnp.float32)
        ix1 = jnp.maximum(bx1, x1)
        iy1 = jnp.maximum(by1, y1)
        ix2 = jnp.minimum(bx2, x2)
        iy2 = jnp.minimum(by2, y2)
        inter = jnp.maximum(ix2 - ix1, 0.0) * jnp.maximum(iy2 - iy1, 0.0)
        a1 = (bx2 - bx1) * (by2 - by1)
        iou = inter / jnp.maximum(a1 + area - inter, 1e-6)
        sc = jnp.where(iou > _IOU_THR, jnp.float32(-1e31), sc)
        hot = slot == t
        keep = jnp.where(hot, idx, keep)
        valid = jnp.where(hot, ok, valid)
        return sc, keep, valid

    def step(t, carry):
        return tuple(one_step(t, *carry[b], b) for b in range(_B))

    zk = jnp.zeros(_OUT_SHAPE, jnp.int32)
    zv = jnp.zeros(_OUT_SHAPE, jnp.float32)
    init = tuple((sc_ref[b], zk, zv) for b in range(_B))
    final = jax.lax.fori_loop(0, _POST_NMS, step, init)
    for b in range(_B):
        keep_ref[b] = final[b][1]
        valid_ref[b] = final[b][2]


def _nms_pallas(boxes, scores):
    n = scores.shape[1]
    pad = _NMS_PAD - n
    sc = jnp.concatenate(
        [scores, jnp.full((_B, pad), -3e31, jnp.float32)], axis=1)
    bx = jnp.concatenate(
        [boxes, jnp.zeros((_B, pad, 4), jnp.float32)], axis=1)
    sc = sc.reshape(_B, *_NMS_SHAPE)
    planes = [bx[:, :, i].reshape(_B, *_NMS_SHAPE) for i in range(4)]
    keep, valid = pl.pallas_call(
        _nms_kernel_body,
        out_shape=[
            jax.ShapeDtypeStruct((_B, *_OUT_SHAPE), jnp.int32),
            jax.ShapeDtypeStruct((_B, *_OUT_SHAPE), jnp.float32),
        ],
    )(sc, *planes)
    return keep.reshape(_B, _POST_NMS), valid.reshape(_B, _POST_NMS)


def kernel(feat0, feat1, feat2, feat3, feat4, rpn_conv_w, rpn_conv_b,
           rpn_cls_w, rpn_cls_b, rpn_box_w, rpn_box_b, fc1_w, fc1_b,
           fc2_w, fc2_b, cls_w, cls_b, reg_w, reg_b):
    feats = [feat0, feat1, feat2, feat3, feat4]
    w9 = rpn_conv_w.transpose(2, 3, 1, 0).reshape(9, _C, _C).astype(jnp.bfloat16)
    wh = jnp.concatenate([rpn_cls_w[:, :, 0, 0], rpn_box_w[:, :, 0, 0]], 0).T
    wh = jnp.pad(wh, ((0, 0), (0, 128 - 15))).astype(jnp.bfloat16)
    hb = jnp.pad(jnp.concatenate([rpn_cls_b, rpn_box_b]), (0, 128 - 15))[None]
    cb = rpn_conv_b[None]
    rpn_cls, rpn_box, flats = [], [], []
    for l, f in enumerate(feats):
        cls_l, box_l, xf_l = _CONV_RUNS[l](f, w9, wh, cb, hb)
        rpn_cls.append(cls_l)
        rpn_box.append(box_l)
        flats.append(xf_l)
    anchors = [_make_anchors(h, w, s)
               for (h, w), s in zip(_FEAT_HW, _STRIDES)]

    per_img_boxes, per_img_scores = [], []
    for b in range(_B):
        lvl_boxes, lvl_scores = [], []
        for l in range(5):
            cls = rpn_cls[l][b].reshape(-1)
            box = rpn_box[l][b].reshape(-1, 4)
            scores = jax.nn.sigmoid(cls)
            k = min(_PRE_NMS_TOPK, scores.shape[0])
            topv, topi = jax.lax.top_k(scores, k)
            lvl_boxes.append(_decode(anchors[l][topi], box[topi]))
            lvl_scores.append(topv)
        per_img_boxes.append(jnp.concatenate(lvl_boxes, 0))
        per_img_scores.append(jnp.concatenate(lvl_scores, 0))

    boxes_cat = jnp.stack(per_img_boxes, 0)
    scores_cat = jnp.stack(per_img_scores, 0)
    keep, valid = _nms_pallas(boxes_cat, scores_cat)

    all_boxes, all_scores = [], []
    for b in range(_B):
        vm = valid[b]
        all_boxes.append(boxes_cat[b][keep[b]] * vm[:, None])
        all_scores.append(scores_cat[b][keep[b]] * vm)

    boxes_all = jnp.stack(all_boxes, 0)
    idx, w4 = _roi_indices(boxes_all)
    arena = jnp.concatenate([f.reshape(-1, _C) for f in flats[:4]], 0)
    g = _sc_gather(arena, idx)
    roi_bf = _combine_call(g.reshape(4, _NPTS, _C), w4)
    roi2 = roi_bf.reshape(_B * _POST_NMS, _POOL * _POOL * _C)
    w1p = (fc1_w.reshape(_FC_DIM, _C, _POOL, _POOL).transpose(0, 2, 3, 1)
           .reshape(_FC_DIM, -1).T.astype(jnp.bfloat16))
    w2t = fc2_w.T.astype(jnp.bfloat16)
    wct = jnp.pad(cls_w, ((0, 128 - 81), (0, 0))).T.astype(jnp.bfloat16)
    wrt = jnp.pad(reg_w, ((0, 384 - 320), (0, 0))).T.astype(jnp.bfloat16)
    bc = jnp.pad(cls_b, (0, 128 - 81))[None]
    br = jnp.pad(reg_b, (0, 384 - 320))[None]
    oc, orr = _fc_call(roi2, w1p, fc1_b[None], w2t, fc2_b[None],
                       wct, bc, wrt, br)
    cls_score = oc[:, :_NUM_CLASSES + 1]
    bbox_pred = orr[:, :_NUM_CLASSES * 4]
    return cls_score, bbox_pred, boxes_all, jnp.stack(all_scores, 0)
